# baseline (speedup 1.0000x reference)
import functools

import jax
import jax.numpy as jnp
from jax import lax
from jax.experimental import pallas as pl
from jax.experimental.pallas import tpu as pltpu
from jax.experimental.pallas import tpu_sc as plsc

L = 16
NC, NS = 2, 16
NW = NC * NS

F = 26
NBINS = 256
GR = 8
GELEMS = F * GR
VPG = GELEMS // L
CH_G = 93
CH_ELEMS = CH_G * GELEMS
CH_ROWS = CH_G * GR
HIST = F * NBINS
HROWS, HCOLS = 64, NBINS

_GDN = lax.GatherDimensionNumbers(
    offset_dims=(), collapsed_slice_dims=(0,), start_index_map=(0,))


def _vgather(vec, idx):
    return lax.gather(vec, idx[:, None], _GDN, (1,),
                      mode=lax.GatherScatterMode.PROMISE_IN_BOUNDS)


def _make_hist(n_rows):
    tot_g = n_rows // GR
    base_g, extra = tot_g // NW, tot_g % NW
    nch = base_g // CH_G
    base_g1, extra1 = tot_g // NS, tot_g % NS
    nch1 = base_g1 // CH_G
    assert base_g == nch * CH_G and nch % 2 == 0
    assert base_g1 == nch1 * CH_G and nch1 % 2 == 0

    mesh = plsc.VectorSubcoreMesh(core_axis_name="c", subcore_axis_name="s")

    @functools.partial(
        pl.kernel,
        out_type=jax.ShapeDtypeStruct((NC, HROWS, HCOLS), jnp.float32),
        mesh=mesh,
        compiler_params=pltpu.CompilerParams(needs_layout_passes=False),
        scratch_types=[
            pltpu.VMEM((CH_ELEMS,), jnp.float32),
            pltpu.VMEM((CH_ELEMS,), jnp.float32),
            pltpu.VMEM((CH_ROWS + L,), jnp.int32),
            pltpu.VMEM((CH_ROWS + L,), jnp.int32),
            pltpu.VMEM((2 * GELEMS,), jnp.float32),
            pltpu.VMEM((NS, 2 * GELEMS), jnp.float32),
            pltpu.VMEM((GELEMS,), jnp.float32),
            pltpu.VMEM((GELEMS,), jnp.float32),
            pltpu.VMEM((7 * GELEMS,), jnp.int32),
            pltpu.VMEM((GELEMS,), jnp.int32),
            pltpu.VMEM((GELEMS,), jnp.int32),
            pltpu.VMEM((HROWS, HCOLS), jnp.float32),
            pltpu.VMEM((HROWS // NS, HCOLS), jnp.float32),
            pltpu.VMEM((HROWS // NS, HCOLS), jnp.float32),
            pltpu.VMEM_SHARED((NS, 2 * GELEMS), jnp.float32),
            pltpu.VMEM_SHARED((NS, HROWS, HCOLS), jnp.float32),
            pltpu.SemaphoreType.DMA,
            pltpu.SemaphoreType.DMA,
            pltpu.SemaphoreType.DMA,
            pltpu.SemaphoreType.DMA,
        ],
    )
    def hist_kernel(data_hbm, labels_hbm, rot_hbm, frow_hbm, rowpat_hbm,
                    out_hbm, buf0, buf1, lbuf0, lbuf1, mmpart, mmb,
                    slotmn, slotmx, rotb, frowb, rowpatb, hist, stripe,
                    stripe2, smm, shist, sem0, sem1, lsem0, lsem1):
        cid = lax.axis_index("c")
        sid = lax.axis_index("s")
        wid = cid * NS + sid

        bufs = (buf0, buf1)
        lbufs = (lbuf0, lbuf1)
        sems = (sem0, sem1)
        lsems = (lsem0, lsem1)

        pltpu.sync_copy(rot_hbm, rotb)
        pltpu.sync_copy(frow_hbm, frowb)
        pltpu.sync_copy(rowpat_hbm, rowpatb)

        my1_g = sid * base_g1 + jnp.minimum(sid, extra1)
        ebase1 = my1_g * GELEMS

        def start1(c, b):
            pltpu.async_copy(
                data_hbm.at[pl.ds(ebase1 + c * CH_ELEMS, CH_ELEMS)],
                bufs[b], sems[b])

        def dwait(b):
            pltpu.make_async_copy(data_hbm.at[pl.ds(0, CH_ELEMS)],
                                  bufs[b], sems[b]).wait()

        start1(0, 0)
        start1(1, 1)

        inf = jnp.full((L,), jnp.inf, jnp.float32)
        ninf = jnp.full((L,), -jnp.inf, jnp.float32)
        accs0 = tuple([inf] * VPG + [ninf] * VPG)

        def mm_chunk(c, accs):
            for b in (0, 1):
                dwait(b)
                buf = bufs[b]

                def group_body(g, accs):
                    mns = list(accs[:VPG])
                    mxs = list(accs[VPG:])
                    gb = g * GELEMS
                    for v in range(VPG):
                        x = buf[pl.ds(gb + v * L, L)]
                        mns[v] = jnp.minimum(mns[v], x)
                        mxs[v] = jnp.maximum(mxs[v], x)
                    return tuple(mns + mxs)

                accs = pl.loop(0, CH_G, init_carry=accs)(group_body)

                @pl.when(c + b + 2 < nch1)
                def _():
                    start1(c + b + 2, b)
            return accs

        accs = pl.loop(0, nch1, step=2, init_carry=accs0)(mm_chunk)

        for v in range(VPG):
            mmpart[pl.ds(v * L, L)] = accs[v]
            mmpart[pl.ds(GELEMS + v * L, L)] = accs[VPG + v]

        if extra1:
            @pl.when(sid < extra1)
            def _():
                xg = my1_g + base_g1
                pltpu.sync_copy(data_hbm.at[pl.ds(xg * GELEMS, GELEMS)],
                                buf0.at[pl.ds(0, GELEMS)])
                for v in range(VPG):
                    x = buf0[pl.ds(v * L, L)]
                    mmpart[pl.ds(v * L, L)] = jnp.minimum(
                        mmpart[pl.ds(v * L, L)], x)
                    mmpart[pl.ds(GELEMS + v * L, L)] = jnp.maximum(
                        mmpart[pl.ds(GELEMS + v * L, L)], x)

        pltpu.sync_copy(mmpart, smm.at[sid])

        my_g = wid * base_g + jnp.minimum(wid, extra)
        ebase = my_g * GELEMS
        rbase = my_g * GR

        def start2(c, b):
            pltpu.async_copy(
                data_hbm.at[pl.ds(ebase + c * CH_ELEMS, CH_ELEMS)],
                bufs[b], sems[b])
            pltpu.async_copy(
                labels_hbm.at[pl.ds(rbase + c * CH_ROWS, CH_ROWS)],
                lbufs[b].at[pl.ds(0, CH_ROWS)], lsems[b])

        def wait2(b):
            dwait(b)
            pltpu.make_async_copy(labels_hbm.at[pl.ds(0, CH_ROWS)],
                                  lbufs[b].at[pl.ds(0, CH_ROWS)],
                                  lsems[b]).wait()

        zero = jnp.zeros((L,), jnp.float32)

        def zrow(r):
            for j in range(HCOLS // L):
                hist[r, pl.ds(j * L, L)] = zero

        pl.loop(0, HROWS)(zrow)

        plsc.subcore_barrier()
        pltpu.sync_copy(smm, mmb)

        start2(0, 0)
        start2(1, 1)

        smn = [mmb[0, pl.ds(v * L, L)] for v in range(VPG)]
        smx = [mmb[0, pl.ds(GELEMS + v * L, L)] for v in range(VPG)]

        def fold_tile(t, carry):
            mns, mxs = carry
            mns = [jnp.minimum(mns[v], mmb[t, pl.ds(v * L, L)])
                   for v in range(VPG)]
            mxs = [jnp.maximum(mxs[v], mmb[t, pl.ds(GELEMS + v * L, L)])
                   for v in range(VPG)]
            return mns, mxs

        smn, smx = pl.loop(1, NS, init_carry=(smn, smx))(fold_tile)
        for v in range(VPG):
            slotmn[pl.ds(v * L, L)] = smn[v]
            slotmx[pl.ds(v * L, L)] = smx[v]
        for j in range(7):
            for v in range(VPG):
                rv = rotb[pl.ds(j * GELEMS + v * L, L)]
                smn[v] = jnp.minimum(smn[v], plsc.load_gather(slotmn, [rv]))
                smx[v] = jnp.maximum(smx[v], plsc.load_gather(slotmx, [rv]))

        one = jnp.ones((L,), jnp.float32)
        mins = smn
        ws = []
        for v in range(VPG):
            wv = (smx[v] - smn[v]) * (1.0 / NBINS)
            wv = jnp.where(wv <= 0.0, one, wv)
            ws.append(one / wv)

        frows = [frowb[pl.ds(v * L, L)] for v in range(VPG)]
        rowpats = [rowpatb[pl.ds(v * L, L)] for v in range(VPG)]
        ones = jnp.ones((L,), jnp.float32)
        maxbin = jnp.full((L,), NBINS - 1, jnp.int32)
        fsplat = jnp.full((L,), F, jnp.int32)

        def do_group(buf, lraw, gb):
            lvec = fsplat - lraw * F
            for v in range(VPG):
                x = buf[pl.ds(gb + v * L, L)]
                lab = _vgather(lvec, rowpats[v])
                b = jnp.minimum(((x - mins[v]) * ws[v]).astype(jnp.int32),
                                maxbin)
                plsc.addupdate_scatter(hist, [frows[v] + lab, b], ones)

        def chunk_body(c):
            for b in (0, 1):
                wait2(b)
                buf = bufs[b]
                lbuf = lbufs[b]

                def group_body(g):
                    lvec = lbuf[pl.ds(g * GR, L)]
                    do_group(buf, lvec, g * GELEMS)

                plsc.parallel_loop(0, CH_G, unroll=3)(group_body)

                @pl.when(c + b + 2 < nch)
                def _():
                    start2(c + b + 2, b)

        pl.loop(0, nch, step=2)(chunk_body)

        if extra:
            @pl.when(wid < extra)
            def _():
                xg = my_g + base_g
                pltpu.sync_copy(data_hbm.at[pl.ds(xg * GELEMS, GELEMS)],
                                buf0.at[pl.ds(0, GELEMS)])
                pltpu.sync_copy(labels_hbm.at[pl.ds(xg * GR, GR)],
                                lbuf0.at[pl.ds(0, GR)])
                do_group(buf0, lbuf0[pl.ds(0, L)], 0)

        SR = HROWS // NS
        pltpu.sync_copy(hist, shist.at[sid])
        plsc.subcore_barrier()

        r0 = sid * SR
        pltpu.sync_copy(shist.at[0, pl.ds(r0, SR)], stripe)

        def acc_tile(t):
            pltpu.sync_copy(shist.at[t, pl.ds(r0, SR)], stripe2)
            for r in range(SR):
                for j in range(HCOLS // L):
                    sl = pl.ds(j * L, L)
                    stripe[r, sl] = stripe[r, sl] + stripe2[r, sl]

        for t in range(1, NS):
            acc_tile(t)

        pltpu.sync_copy(stripe, out_hbm.at[cid, pl.ds(r0, SR)])

    return hist_kernel


def kernel(data, labels, n_bins):
    n_rows, f = data.shape
    assert f == F and n_rows % GR == 0
    del n_bins

    data1d = data.reshape(-1)

    kk = jnp.arange(GELEMS, dtype=jnp.int32)
    frow208 = kk % F
    rowpat208 = kk // F
    jj = jnp.arange(7 * GELEMS, dtype=jnp.int32)
    rot = (jj % GELEMS + F * (jj // GELEMS + 1)) % GELEMS

    parts = _make_hist(n_rows)(data1d, labels, rot, frow208, rowpat208)
    flat = (parts[0] + parts[1]).reshape(HROWS * HCOLS)[:2 * HIST]
    return flat.reshape(2, F, NBINS)


# speedup vs baseline: 1.1321x; 1.1321x over previous
import functools

import jax
import jax.numpy as jnp
from jax import lax
from jax.experimental import pallas as pl
from jax.experimental.pallas import tpu as pltpu
from jax.experimental.pallas import tpu_sc as plsc

L = 16
NC, NS = 2, 16
NW = NC * NS

F = 26
NBINS = 256
GR = 8
GELEMS = F * GR
VPG = GELEMS // L
CH_G = 93
CH_ELEMS = CH_G * GELEMS
CH_ROWS = CH_G * GR
HIST = F * NBINS
HROWS, HCOLS = 64, NBINS

_GDN = lax.GatherDimensionNumbers(
    offset_dims=(), collapsed_slice_dims=(0,), start_index_map=(0,))


def _vgather(vec, idx):
    return lax.gather(vec, idx[:, None], _GDN, (1,),
                      mode=lax.GatherScatterMode.PROMISE_IN_BOUNDS)


def _split(n_rows):
    tot_g = n_rows // GR
    return tot_g // NW, tot_g % NW


def _make_minmax(n_rows):
    base_g, extra = _split(n_rows)
    nch = base_g // CH_G
    assert base_g == nch * CH_G and nch % 2 == 0

    mesh = plsc.VectorSubcoreMesh(core_axis_name="c", subcore_axis_name="s")

    @functools.partial(
        pl.kernel,
        out_type=jax.ShapeDtypeStruct((NW, 2 * GELEMS), jnp.float32),
        mesh=mesh,
        compiler_params=pltpu.CompilerParams(needs_layout_passes=False),
        scratch_types=[
            pltpu.VMEM((CH_ELEMS,), jnp.float32),
            pltpu.VMEM((CH_ELEMS,), jnp.float32),
            pltpu.VMEM((2 * GELEMS,), jnp.float32),
            pltpu.SemaphoreType.DMA,
            pltpu.SemaphoreType.DMA,
        ],
    )
    def minmax_kernel(data_hbm, out_hbm, buf0, buf1, mmbuf, sem0, sem1):
        wid = lax.axis_index("c") * NS + lax.axis_index("s")
        my_base_g = wid * base_g + jnp.minimum(wid, extra)
        ebase = my_base_g * GELEMS

        bufs = (buf0, buf1)
        sems = (sem0, sem1)

        def start(c, b):
            pltpu.async_copy(
                data_hbm.at[pl.ds(ebase + c * CH_ELEMS, CH_ELEMS)],
                bufs[b], sems[b])

        def wait(b):
            pltpu.make_async_copy(data_hbm.at[pl.ds(0, CH_ELEMS)],
                                  bufs[b], sems[b]).wait()

        start(0, 0)
        start(1, 1)

        inf = jnp.full((L,), jnp.inf, jnp.float32)
        ninf = jnp.full((L,), -jnp.inf, jnp.float32)
        accs0 = tuple([inf] * VPG + [ninf] * VPG)

        def chunk_body(c, accs):
            for b in (0, 1):
                wait(b)
                buf = bufs[b]

                def group_body(g, accs):
                    mns = list(accs[:VPG])
                    mxs = list(accs[VPG:])
                    gb = g * GELEMS
                    for v in range(VPG):
                        x = buf[pl.ds(gb + v * L, L)]
                        mns[v] = jnp.minimum(mns[v], x)
                        mxs[v] = jnp.maximum(mxs[v], x)
                    return tuple(mns + mxs)

                accs = pl.loop(0, CH_G, init_carry=accs)(group_body)

                @pl.when(c + b + 2 < nch)
                def _():
                    start(c + b + 2, b)
            return accs

        accs = pl.loop(0, nch, step=2, init_carry=accs0)(chunk_body)

        for v in range(VPG):
            mmbuf[pl.ds(v * L, L)] = accs[v]
            mmbuf[pl.ds(GELEMS + v * L, L)] = accs[VPG + v]

        if extra:
            @pl.when(wid < extra)
            def _():
                xg = my_base_g + base_g
                pltpu.sync_copy(data_hbm.at[pl.ds(xg * GELEMS, GELEMS)],
                                buf0.at[pl.ds(0, GELEMS)])
                for v in range(VPG):
                    x = buf0[pl.ds(v * L, L)]
                    mmbuf[pl.ds(v * L, L)] = jnp.minimum(
                        mmbuf[pl.ds(v * L, L)], x)
                    mmbuf[pl.ds(GELEMS + v * L, L)] = jnp.maximum(
                        mmbuf[pl.ds(GELEMS + v * L, L)], x)

        pltpu.sync_copy(mmbuf, out_hbm.at[wid])

    return minmax_kernel


def _make_hist(n_rows):
    base_g, extra = _split(n_rows)
    nch = base_g // CH_G

    mesh = plsc.VectorSubcoreMesh(core_axis_name="c", subcore_axis_name="s")

    @functools.partial(
        pl.kernel,
        out_type=jax.ShapeDtypeStruct((NC, HROWS * HCOLS), jnp.float32),
        mesh=mesh,
        compiler_params=pltpu.CompilerParams(needs_layout_passes=False),
        scratch_types=[
            pltpu.VMEM((CH_ELEMS,), jnp.float32),
            pltpu.VMEM((CH_ELEMS,), jnp.float32),
            pltpu.VMEM((CH_ROWS + L,), jnp.int32),
            pltpu.VMEM((CH_ROWS + L,), jnp.int32),
            pltpu.VMEM((NW, 2 * GELEMS), jnp.float32),
            pltpu.VMEM((GELEMS,), jnp.float32),
            pltpu.VMEM((GELEMS,), jnp.float32),
            pltpu.VMEM((7 * GELEMS,), jnp.int32),
            pltpu.VMEM((GELEMS,), jnp.int32),
            pltpu.VMEM((GELEMS,), jnp.int32),
            pltpu.VMEM((HROWS * HCOLS,), jnp.float32),
            pltpu.VMEM((HROWS * HCOLS // NS,), jnp.float32),
            pltpu.VMEM((HROWS * HCOLS // NS,), jnp.float32),
            pltpu.VMEM_SHARED((NS, HROWS * HCOLS), jnp.float32),
            pltpu.SemaphoreType.DMA,
            pltpu.SemaphoreType.DMA,
            pltpu.SemaphoreType.DMA,
            pltpu.SemaphoreType.DMA,
        ],
    )
    def hist_kernel(data_hbm, labels_hbm, mm_hbm, rot_hbm, frow_hbm,
                    rowpat_hbm, out_hbm, buf0, buf1, lbuf0, lbuf1, mmb,
                    slotmn, slotmx, rotb, frowb, rowpatb, hist, stripe,
                    stripe2, shist, sem0, sem1, lsem0, lsem1):
        cid = lax.axis_index("c")
        sid = lax.axis_index("s")
        wid = cid * NS + sid
        my_base_g = wid * base_g + jnp.minimum(wid, extra)
        ebase = my_base_g * GELEMS
        rbase = my_base_g * GR

        bufs = (buf0, buf1)
        lbufs = (lbuf0, lbuf1)
        sems = (sem0, sem1)
        lsems = (lsem0, lsem1)

        def start(c, b):
            pltpu.async_copy(
                data_hbm.at[pl.ds(ebase + c * CH_ELEMS, CH_ELEMS)],
                bufs[b], sems[b])
            pltpu.async_copy(
                labels_hbm.at[pl.ds(rbase + c * CH_ROWS, CH_ROWS)],
                lbufs[b].at[pl.ds(0, CH_ROWS)], lsems[b])

        def wait(b):
            pltpu.make_async_copy(data_hbm.at[pl.ds(0, CH_ELEMS)],
                                  bufs[b], sems[b]).wait()
            pltpu.make_async_copy(labels_hbm.at[pl.ds(0, CH_ROWS)],
                                  lbufs[b].at[pl.ds(0, CH_ROWS)],
                                  lsems[b]).wait()

        pltpu.sync_copy(mm_hbm, mmb)
        pltpu.sync_copy(rot_hbm, rotb)
        pltpu.sync_copy(frow_hbm, frowb)
        pltpu.sync_copy(rowpat_hbm, rowpatb)

        start(0, 0)
        start(1, 1)

        zero = jnp.zeros((L,), jnp.float32)

        def zrow(r):
            for j in range(4):
                hist[pl.ds(r * 64 + j * L, L)] = zero

        pl.loop(0, HROWS * HCOLS // 64)(zrow)

        iota = lax.iota(jnp.int32, L)
        smn = [mmb[0, pl.ds(v * L, L)] for v in range(VPG)]
        smx = [mmb[0, pl.ds(GELEMS + v * L, L)] for v in range(VPG)]

        def fold_tile(t, carry):
            mns, mxs = carry
            mns = [jnp.minimum(mns[v], mmb[t, pl.ds(v * L, L)])
                   for v in range(VPG)]
            mxs = [jnp.maximum(mxs[v], mmb[t, pl.ds(GELEMS + v * L, L)])
                   for v in range(VPG)]
            return mns, mxs

        smn, smx = pl.loop(1, NW, init_carry=(smn, smx))(fold_tile)
        for v in range(VPG):
            slotmn[pl.ds(v * L, L)] = smn[v]
            slotmx[pl.ds(v * L, L)] = smx[v]
        for j in range(7):
            for v in range(VPG):
                rv = rotb[pl.ds(j * GELEMS + v * L, L)]
                smn[v] = jnp.minimum(smn[v], plsc.load_gather(slotmn, [rv]))
                smx[v] = jnp.maximum(smx[v], plsc.load_gather(slotmx, [rv]))

        one = jnp.ones((L,), jnp.float32)
        mins = smn
        ws = []
        for v in range(VPG):
            wv = (smx[v] - smn[v]) * (1.0 / NBINS)
            wv = jnp.where(wv <= 0.0, one, wv)
            ws.append(one / wv)

        frows = [frowb[pl.ds(v * L, L)] for v in range(VPG)]
        rowpats = [rowpatb[pl.ds(v * L, L)] for v in range(VPG)]
        ones = jnp.ones((L,), jnp.float32)
        maxbin = jnp.full((L,), NBINS - 1, jnp.int32)
        hsplat = jnp.full((L,), HIST, jnp.int32)

        def do_group(buf, lraw, gb):
            lvec = hsplat - lraw * HIST
            for v in range(VPG):
                x = buf[pl.ds(gb + v * L, L)]
                lab = _vgather(lvec, rowpats[v])
                b = jnp.minimum(((x - mins[v]) * ws[v]).astype(jnp.int32),
                                maxbin)
                plsc.addupdate_scatter(hist, [b + (frows[v] + lab)], ones)

        def chunk_body(c):
            for b in (0, 1):
                wait(b)
                buf = bufs[b]
                lbuf = lbufs[b]

                def group_body(g):
                    lvec = lbuf[pl.ds(g * GR, L)]
                    do_group(buf, lvec, g * GELEMS)

                plsc.parallel_loop(0, CH_G, unroll=3)(group_body)

                @pl.when(c + b + 2 < nch)
                def _():
                    start(c + b + 2, b)

        pl.loop(0, nch, step=2)(chunk_body)

        if extra:
            @pl.when(wid < extra)
            def _():
                xg = my_base_g + base_g
                pltpu.sync_copy(data_hbm.at[pl.ds(xg * GELEMS, GELEMS)],
                                buf0.at[pl.ds(0, GELEMS)])
                pltpu.sync_copy(labels_hbm.at[pl.ds(xg * GR, GR)],
                                lbuf0.at[pl.ds(0, GR)])
                do_group(buf0, lbuf0[pl.ds(0, L)], 0)

        SE = HROWS * HCOLS // NS
        pltpu.sync_copy(hist, shist.at[sid])
        plsc.subcore_barrier()

        e0 = sid * SE
        pltpu.sync_copy(shist.at[0, pl.ds(e0, SE)], stripe)

        def acc_tile(t):
            pltpu.sync_copy(shist.at[t, pl.ds(e0, SE)], stripe2)
            for j in range(SE // L):
                sl = pl.ds(j * L, L)
                stripe[sl] = stripe[sl] + stripe2[sl]

        for t in range(1, NS):
            acc_tile(t)

        pltpu.sync_copy(stripe, out_hbm.at[cid, pl.ds(e0, SE)])

    return hist_kernel


def kernel(data, labels, n_bins):
    n_rows, f = data.shape
    assert f == F and n_rows % GR == 0

    del n_bins

    data1d = data.reshape(-1)
    mm = _make_minmax(n_rows)(data1d)

    kk = jnp.arange(GELEMS, dtype=jnp.int32)
    frow208 = (kk % F) * NBINS
    rowpat208 = kk // F
    jj = jnp.arange(7 * GELEMS, dtype=jnp.int32)
    rot = (jj % GELEMS + F * (jj // GELEMS + 1)) % GELEMS

    parts = _make_hist(n_rows)(data1d, labels, mm, rot, frow208, rowpat208)
    flat = (parts[0] + parts[1]).reshape(HROWS * HCOLS)[:2 * HIST]
    return flat.reshape(2, F, NBINS)


# speedup vs baseline: 1.1325x; 1.0004x over previous
import functools

import jax
import jax.numpy as jnp
from jax import lax
from jax.experimental import pallas as pl
from jax.experimental.pallas import tpu as pltpu
from jax.experimental.pallas import tpu_sc as plsc

L = 16
NC, NS = 2, 16
NW = NC * NS

F = 26
NBINS = 256
GR = 8
GELEMS = F * GR
VPG = GELEMS // L
CH_G = 93
CH_ELEMS = CH_G * GELEMS
CH_ROWS = CH_G * GR
HIST = F * NBINS
HROWS, HCOLS = 64, NBINS

_GDN = lax.GatherDimensionNumbers(
    offset_dims=(), collapsed_slice_dims=(0,), start_index_map=(0,))


def _vgather(vec, idx):
    return lax.gather(vec, idx[:, None], _GDN, (1,),
                      mode=lax.GatherScatterMode.PROMISE_IN_BOUNDS)


def _split(n_rows):
    tot_g = n_rows // GR
    return tot_g // NW, tot_g % NW


def _make_minmax(n_rows):
    base_g, extra = _split(n_rows)
    nch = base_g // CH_G
    assert base_g == nch * CH_G and nch % 2 == 0

    mesh = plsc.VectorSubcoreMesh(core_axis_name="c", subcore_axis_name="s")

    @functools.partial(
        pl.kernel,
        out_type=jax.ShapeDtypeStruct((NW, 2 * GELEMS), jnp.float32),
        mesh=mesh,
        compiler_params=pltpu.CompilerParams(needs_layout_passes=False),
        scratch_types=[
            pltpu.VMEM((CH_ELEMS,), jnp.float32),
            pltpu.VMEM((CH_ELEMS,), jnp.float32),
            pltpu.VMEM((2 * GELEMS,), jnp.float32),
            pltpu.SemaphoreType.DMA,
            pltpu.SemaphoreType.DMA,
        ],
    )
    def minmax_kernel(data_hbm, out_hbm, buf0, buf1, mmbuf, sem0, sem1):
        wid = lax.axis_index("c") * NS + lax.axis_index("s")
        my_base_g = wid * base_g + jnp.minimum(wid, extra)
        ebase = my_base_g * GELEMS

        bufs = (buf0, buf1)
        sems = (sem0, sem1)

        def start(c, b):
            pltpu.async_copy(
                data_hbm.at[pl.ds(ebase + c * CH_ELEMS, CH_ELEMS)],
                bufs[b], sems[b])

        def wait(b):
            pltpu.make_async_copy(data_hbm.at[pl.ds(0, CH_ELEMS)],
                                  bufs[b], sems[b]).wait()

        start(0, 0)
        start(1, 1)

        inf = jnp.full((L,), jnp.inf, jnp.float32)
        ninf = jnp.full((L,), -jnp.inf, jnp.float32)
        accs0 = tuple([inf] * VPG + [ninf] * VPG)

        def chunk_body(c, accs):
            for b in (0, 1):
                wait(b)
                buf = bufs[b]

                def group_body(g, accs):
                    mns = list(accs[:VPG])
                    mxs = list(accs[VPG:])
                    gb = g * GELEMS
                    for v in range(VPG):
                        x = buf[pl.ds(gb + v * L, L)]
                        mns[v] = jnp.minimum(mns[v], x)
                        mxs[v] = jnp.maximum(mxs[v], x)
                    return tuple(mns + mxs)

                accs = pl.loop(0, CH_G, init_carry=accs)(group_body)

                @pl.when(c + b + 2 < nch)
                def _():
                    start(c + b + 2, b)
            return accs

        accs = pl.loop(0, nch, step=2, init_carry=accs0)(chunk_body)

        for v in range(VPG):
            mmbuf[pl.ds(v * L, L)] = accs[v]
            mmbuf[pl.ds(GELEMS + v * L, L)] = accs[VPG + v]

        if extra:
            @pl.when(wid < extra)
            def _():
                xg = my_base_g + base_g
                pltpu.sync_copy(data_hbm.at[pl.ds(xg * GELEMS, GELEMS)],
                                buf0.at[pl.ds(0, GELEMS)])
                for v in range(VPG):
                    x = buf0[pl.ds(v * L, L)]
                    mmbuf[pl.ds(v * L, L)] = jnp.minimum(
                        mmbuf[pl.ds(v * L, L)], x)
                    mmbuf[pl.ds(GELEMS + v * L, L)] = jnp.maximum(
                        mmbuf[pl.ds(GELEMS + v * L, L)], x)

        pltpu.sync_copy(mmbuf, out_hbm.at[wid])

    return minmax_kernel


def _make_hist(n_rows):
    base_g, extra = _split(n_rows)
    nch = base_g // CH_G

    mesh = plsc.VectorSubcoreMesh(core_axis_name="c", subcore_axis_name="s")

    @functools.partial(
        pl.kernel,
        out_type=jax.ShapeDtypeStruct((NC, HROWS * HCOLS), jnp.float32),
        mesh=mesh,
        compiler_params=pltpu.CompilerParams(needs_layout_passes=False),
        scratch_types=[
            pltpu.VMEM((CH_ELEMS,), jnp.float32),
            pltpu.VMEM((CH_ELEMS,), jnp.float32),
            pltpu.VMEM((CH_ROWS + L,), jnp.int32),
            pltpu.VMEM((CH_ROWS + L,), jnp.int32),
            pltpu.VMEM((NW, 2 * GELEMS), jnp.float32),
            pltpu.VMEM((GELEMS,), jnp.float32),
            pltpu.VMEM((GELEMS,), jnp.float32),
            pltpu.VMEM((7 * GELEMS,), jnp.int32),
            pltpu.VMEM((GELEMS,), jnp.int32),
            pltpu.VMEM((GELEMS,), jnp.int32),
            pltpu.VMEM((HROWS * HCOLS,), jnp.float32),
            pltpu.VMEM((HROWS * HCOLS // NS,), jnp.float32),
            pltpu.VMEM((HROWS * HCOLS // NS,), jnp.float32),
            pltpu.VMEM_SHARED((NS, HROWS * HCOLS), jnp.float32),
            pltpu.SemaphoreType.DMA,
            pltpu.SemaphoreType.DMA,
            pltpu.SemaphoreType.DMA,
            pltpu.SemaphoreType.DMA,
        ],
    )
    def hist_kernel(data_hbm, labels_hbm, mm_hbm, rot_hbm, frow_hbm,
                    rowpat_hbm, out_hbm, buf0, buf1, lbuf0, lbuf1, mmb,
                    slotmn, slotmx, rotb, frowb, rowpatb, hist, stripe,
                    stripe2, shist, sem0, sem1, lsem0, lsem1):
        cid = lax.axis_index("c")
        sid = lax.axis_index("s")
        wid = cid * NS + sid
        my_base_g = wid * base_g + jnp.minimum(wid, extra)
        ebase = my_base_g * GELEMS
        rbase = my_base_g * GR

        bufs = (buf0, buf1)
        lbufs = (lbuf0, lbuf1)
        sems = (sem0, sem1)
        lsems = (lsem0, lsem1)

        def start(c, b):
            pltpu.async_copy(
                data_hbm.at[pl.ds(ebase + c * CH_ELEMS, CH_ELEMS)],
                bufs[b], sems[b])
            pltpu.async_copy(
                labels_hbm.at[pl.ds(rbase + c * CH_ROWS, CH_ROWS)],
                lbufs[b].at[pl.ds(0, CH_ROWS)], lsems[b])

        def wait(b):
            pltpu.make_async_copy(data_hbm.at[pl.ds(0, CH_ELEMS)],
                                  bufs[b], sems[b]).wait()
            pltpu.make_async_copy(labels_hbm.at[pl.ds(0, CH_ROWS)],
                                  lbufs[b].at[pl.ds(0, CH_ROWS)],
                                  lsems[b]).wait()

        pltpu.sync_copy(mm_hbm, mmb)
        pltpu.sync_copy(rot_hbm, rotb)
        pltpu.sync_copy(frow_hbm, frowb)
        pltpu.sync_copy(rowpat_hbm, rowpatb)

        start(0, 0)
        start(1, 1)

        zero = jnp.zeros((L,), jnp.float32)

        def zrow(r):
            for j in range(4):
                hist[pl.ds(r * 64 + j * L, L)] = zero

        pl.loop(0, HROWS * HCOLS // 64)(zrow)

        smn = [mmb[0, pl.ds(v * L, L)] for v in range(VPG)]
        smx = [mmb[0, pl.ds(GELEMS + v * L, L)] for v in range(VPG)]

        def fold_tile(t, carry):
            mns, mxs = carry
            mns = [jnp.minimum(mns[v], mmb[t, pl.ds(v * L, L)])
                   for v in range(VPG)]
            mxs = [jnp.maximum(mxs[v], mmb[t, pl.ds(GELEMS + v * L, L)])
                   for v in range(VPG)]
            return mns, mxs

        smn, smx = pl.loop(1, NW, init_carry=(smn, smx))(fold_tile)
        for v in range(VPG):
            slotmn[pl.ds(v * L, L)] = smn[v]
            slotmx[pl.ds(v * L, L)] = smx[v]
        for j in range(7):
            for v in range(VPG):
                rv = rotb[pl.ds(j * GELEMS + v * L, L)]
                smn[v] = jnp.minimum(smn[v], plsc.load_gather(slotmn, [rv]))
                smx[v] = jnp.maximum(smx[v], plsc.load_gather(slotmx, [rv]))

        one = jnp.ones((L,), jnp.float32)
        mins = smn
        ws = []
        for v in range(VPG):
            wv = (smx[v] - smn[v]) * (1.0 / NBINS)
            wv = jnp.where(wv <= 0.0, one, wv)
            ws.append(one / wv)

        frows = [frowb[pl.ds(v * L, L)] for v in range(VPG)]
        rowpats = [rowpatb[pl.ds(v * L, L)] for v in range(VPG)]
        ones = jnp.ones((L,), jnp.float32)
        maxbin = jnp.full((L,), NBINS - 1, jnp.int32)
        hsplat = jnp.full((L,), HIST, jnp.int32)

        def do_group(buf, lraw, gb):
            lvec = hsplat - lraw * HIST
            for v in range(VPG):
                x = buf[pl.ds(gb + v * L, L)]
                lab = _vgather(lvec, rowpats[v])
                b = jnp.minimum(((x - mins[v]) * ws[v]).astype(jnp.int32),
                                maxbin)
                plsc.addupdate_scatter(hist, [b + (frows[v] + lab)], ones)

        def chunk_body(c):
            for b in (0, 1):
                wait(b)
                buf = bufs[b]
                lbuf = lbufs[b]

                def group_body(g):
                    lvec = lbuf[pl.ds(g * GR, L)]
                    do_group(buf, lvec, g * GELEMS)

                plsc.parallel_loop(0, CH_G, unroll=3)(group_body)

                @pl.when(c + b + 2 < nch)
                def _():
                    start(c + b + 2, b)

        pl.loop(0, nch, step=2)(chunk_body)

        if extra:
            @pl.when(wid < extra)
            def _():
                xg = my_base_g + base_g
                pltpu.sync_copy(data_hbm.at[pl.ds(xg * GELEMS, GELEMS)],
                                buf0.at[pl.ds(0, GELEMS)])
                pltpu.sync_copy(labels_hbm.at[pl.ds(xg * GR, GR)],
                                lbuf0.at[pl.ds(0, GR)])
                do_group(buf0, lbuf0[pl.ds(0, L)], 0)

        SE = HROWS * HCOLS // NS
        pltpu.sync_copy(hist, shist.at[sid])
        plsc.subcore_barrier()

        e0 = sid * SE
        pltpu.sync_copy(shist.at[0, pl.ds(e0, SE)], stripe)

        def acc_tile(t):
            pltpu.sync_copy(shist.at[t, pl.ds(e0, SE)], stripe2)
            for j in range(SE // L):
                sl = pl.ds(j * L, L)
                stripe[sl] = stripe[sl] + stripe2[sl]

        for t in range(1, NS):
            acc_tile(t)

        pltpu.sync_copy(stripe, out_hbm.at[cid, pl.ds(e0, SE)])

    return hist_kernel


def kernel(data, labels, n_bins):
    n_rows, f = data.shape
    assert f == F and n_rows % GR == 0

    del n_bins

    data1d = lax.optimization_barrier(data.reshape(-1))
    mm = _make_minmax(n_rows)(data1d)

    kk = jnp.arange(GELEMS, dtype=jnp.int32)
    frow208 = (kk % F) * NBINS
    rowpat208 = kk // F
    jj = jnp.arange(7 * GELEMS, dtype=jnp.int32)
    rot = (jj % GELEMS + F * (jj // GELEMS + 1)) % GELEMS

    parts = _make_hist(n_rows)(data1d, labels, mm, rot, frow208, rowpat208)
    flat = (parts[0] + parts[1]).reshape(HROWS * HCOLS)[:2 * HIST]
    return flat.reshape(2, F, NBINS)
